# baseline (speedup 1.0000x reference)
import functools

import jax
import jax.numpy as jnp
from jax import lax
from jax.experimental import pallas as pl
from jax.experimental.pallas import tpu as pltpu
from jax.experimental.pallas import tpu_sc as plsc

EMBED = 768
B, S = 4, 2048
NROWS = B * S

NC, NS = 2, 16
NW = NC * NS
SBLK = S // NW
CH = 32
HALVES = SBLK // CH
NCH = B * HALVES
LANES = 16
VECS = EMBED // LANES


def _emb_body(ids_hbm, wte_hbm, wpe_hbm, out_hbm,
              idx_v, g0, g1, g2, pbuf,
              gs0, gs1, gs2, psem, os0, os1, os2):
    wid = lax.axis_index("s") * NC + lax.axis_index("c")
    s_base = wid * SBLK

    for b in range(B):
        pltpu.sync_copy(ids_hbm.at[b, wid],
                        idx_v.at[pl.ds(b * HALVES, HALVES)])
    p_cp = pltpu.async_copy(wpe_hbm.at[pl.ds(s_base, SBLK)], pbuf, psem)

    gbuf = (g0, g1, g2)
    gsem = (gs0, gs1, gs2)
    osem = (os0, os1, os2)

    def gcopy(c):
        bg = c % 3
        return pltpu.async_copy(wte_hbm.at[idx_v.at[c]], gbuf[bg], gsem[bg])

    GROUP = 8
    SUBR = 8
    NSUB = CH // SUBR

    def add_sub(gb, pb):
        def row_body(r, carry):
            for j0 in range(0, VECS, GROUP):
                xs = [pb[r, pl.ds((j0 + k) * LANES, LANES)]
                      for k in range(GROUP)]
                for k in range(GROUP):
                    plsc.addupdate(
                        gb.at[r, pl.ds((j0 + k) * LANES, LANES)], xs[k])
            return carry
        lax.fori_loop(0, SUBR, row_body, 0)

    pending_g = {0: gcopy(0), 1: gcopy(1)}
    out_cp = {}
    for c in range(NCH):
        bg = c % 3
        if c + 2 < NCH:
            if c >= 1:
                for cp in out_cp.pop(c - 1):
                    cp.wait()
            pending_g[c + 2] = gcopy(c + 2)
        pending_g.pop(c).wait()
        if c == 0:
            p_cp.wait()
        off = (c % HALVES) * CH
        row0 = (c // HALVES) * S + (c % HALVES) * CH + s_base
        cps = []
        for sub in range(NSUB):
            gb = gbuf[bg].at[pl.ds(sub * SUBR, SUBR)]
            add_sub(gb, pbuf.at[pl.ds(off + sub * SUBR, SUBR)])
            cps.append(pltpu.async_copy(
                gb, out_hbm.at[pl.ds(row0 + sub * SUBR, SUBR)], osem[bg]))
        out_cp[c] = cps
    for c in sorted(out_cp):
        for cp in out_cp.pop(c):
            cp.wait()


@functools.partial(
    pl.kernel,
    mesh=plsc.VectorSubcoreMesh(core_axis_name="c", subcore_axis_name="s"),
    out_type=jax.ShapeDtypeStruct((NROWS, EMBED), jnp.float32),
    scratch_types=[
        pltpu.VMEM((NCH, CH), jnp.int32),
        pltpu.VMEM((CH, EMBED), jnp.float32),
        pltpu.VMEM((CH, EMBED), jnp.float32),
        pltpu.VMEM((CH, EMBED), jnp.float32),
        pltpu.VMEM((SBLK, EMBED), jnp.float32),
        pltpu.SemaphoreType.DMA,
        pltpu.SemaphoreType.DMA,
        pltpu.SemaphoreType.DMA,
        pltpu.SemaphoreType.DMA,
        pltpu.SemaphoreType.DMA,
        pltpu.SemaphoreType.DMA,
        pltpu.SemaphoreType.DMA,
    ],
)
def _emb(ids_hbm, wte_hbm, wpe_hbm, out_hbm, *scratch):
    _emb_body(ids_hbm, wte_hbm, wpe_hbm, out_hbm, *scratch)


def kernel(input_ids, wte, wpe):
    batch, seq = input_ids.shape
    ids4 = input_ids.astype(jnp.int32).reshape(batch, NW, HALVES, CH)
    out = _emb(ids4, wte, wpe)
    return out.reshape(batch, seq, EMBED)


# speedup vs baseline: 1.0295x; 1.0295x over previous
import functools

import jax
import jax.numpy as jnp
from jax import lax
from jax.experimental import pallas as pl
from jax.experimental.pallas import tpu as pltpu
from jax.experimental.pallas import tpu_sc as plsc

EMBED = 768
B, S = 4, 2048
NROWS = B * S

NC, NS = 2, 16
NW = NC * NS
SBLK = S // NW
CH = 32
HALVES = SBLK // CH
NCH = B * HALVES
LANES = 16
VECS = EMBED // LANES


def _emb_body(ids_hbm, wte_hbm, wpe_hbm, out_hbm,
              idx_v, g0, g1, g2, pbuf,
              gs0, gs1, gs2, psem, os0, os1, os2):
    wid = lax.axis_index("s") * NC + lax.axis_index("c")
    s_base = wid * SBLK

    for b in range(B):
        pltpu.sync_copy(ids_hbm.at[b, wid],
                        idx_v.at[pl.ds(b * HALVES, HALVES)])
    p_cp = pltpu.async_copy(wpe_hbm.at[pl.ds(s_base, SBLK)], pbuf, psem)

    gbuf = (g0, g1, g2)
    gsem = (gs0, gs1, gs2)
    osem = (os0, os1, os2)

    def gcopy(c):
        bg = c % 3
        return pltpu.async_copy(wte_hbm.at[idx_v.at[c]], gbuf[bg], gsem[bg])

    GROUP = 8
    SUBR = 16
    NSUB = CH // SUBR

    def add_sub(gb, pb):
        def row_body(r, carry):
            for j0 in range(0, VECS, GROUP):
                xs = [pb[r, pl.ds((j0 + k) * LANES, LANES)]
                      for k in range(GROUP)]
                for k in range(GROUP):
                    plsc.addupdate(
                        gb.at[r, pl.ds((j0 + k) * LANES, LANES)], xs[k])
            return carry
        lax.fori_loop(0, SUBR, row_body, 0)

    pending_g = {0: gcopy(0), 1: gcopy(1)}
    out_cp = {}
    for c in range(NCH):
        bg = c % 3
        if c + 2 < NCH:
            if c >= 1:
                for cp in out_cp.pop(c - 1):
                    cp.wait()
            pending_g[c + 2] = gcopy(c + 2)
        pending_g.pop(c).wait()
        if c == 0:
            p_cp.wait()
        off = (c % HALVES) * CH
        row0 = (c // HALVES) * S + (c % HALVES) * CH + s_base
        cps = []
        for sub in range(NSUB):
            gb = gbuf[bg].at[pl.ds(sub * SUBR, SUBR)]
            add_sub(gb, pbuf.at[pl.ds(off + sub * SUBR, SUBR)])
            cps.append(pltpu.async_copy(
                gb, out_hbm.at[pl.ds(row0 + sub * SUBR, SUBR)], osem[bg]))
        out_cp[c] = cps
    for c in sorted(out_cp):
        for cp in out_cp.pop(c):
            cp.wait()


@functools.partial(
    pl.kernel,
    mesh=plsc.VectorSubcoreMesh(core_axis_name="c", subcore_axis_name="s"),
    out_type=jax.ShapeDtypeStruct((NROWS, EMBED), jnp.float32),
    scratch_types=[
        pltpu.VMEM((NCH, CH), jnp.int32),
        pltpu.VMEM((CH, EMBED), jnp.float32),
        pltpu.VMEM((CH, EMBED), jnp.float32),
        pltpu.VMEM((CH, EMBED), jnp.float32),
        pltpu.VMEM((SBLK, EMBED), jnp.float32),
        pltpu.SemaphoreType.DMA,
        pltpu.SemaphoreType.DMA,
        pltpu.SemaphoreType.DMA,
        pltpu.SemaphoreType.DMA,
        pltpu.SemaphoreType.DMA,
        pltpu.SemaphoreType.DMA,
        pltpu.SemaphoreType.DMA,
    ],
)
def _emb(ids_hbm, wte_hbm, wpe_hbm, out_hbm, *scratch):
    _emb_body(ids_hbm, wte_hbm, wpe_hbm, out_hbm, *scratch)


def kernel(input_ids, wte, wpe):
    batch, seq = input_ids.shape
    ids4 = input_ids.astype(jnp.int32).reshape(batch, NW, HALVES, CH)
    out = _emb(ids4, wte, wpe)
    return out.reshape(batch, seq, EMBED)


# speedup vs baseline: 1.1060x; 1.0743x over previous
import functools

import jax
import jax.numpy as jnp
from jax import lax
from jax.experimental import pallas as pl
from jax.experimental.pallas import tpu as pltpu
from jax.experimental.pallas import tpu_sc as plsc

EMBED = 768
B, S = 4, 2048
NROWS = B * S

NC, NS = 2, 16
NW = NC * NS
SBLK = S // NW
CH = 32
HALVES = SBLK // CH
NCH = B * HALVES
LANES = 16
VECS = EMBED // LANES


def _emb_body(ids_hbm, wte_hbm, wpe_hbm, out_hbm,
              idx_v, g0, g1, g2, pbuf,
              gs0, gs1, gs2, psem, os0, os1, os2, isem):
    wid = lax.axis_index("s") * NC + lax.axis_index("c")
    s_base = wid * SBLK

    idx_cp = [pltpu.async_copy(ids_hbm.at[b, wid],
                               idx_v.at[pl.ds(b * HALVES, HALVES)], isem)
              for b in range(B)]
    p_cp = pltpu.async_copy(wpe_hbm.at[pl.ds(s_base, SBLK)], pbuf, psem)

    gbuf = (g0, g1, g2)
    gsem = (gs0, gs1, gs2)
    osem = (os0, os1, os2)

    for cp in idx_cp:
        cp.wait()

    def gcopy(c):
        bg = c % 3
        return pltpu.async_copy(wte_hbm.at[idx_v.at[c]], gbuf[bg], gsem[bg])

    GROUP = 8
    SUBR = CH
    NSUB = CH // SUBR

    def add_sub(gb, pb):
        def row_body(r, carry):
            for j0 in range(0, VECS, GROUP):
                xs = [pb[r, pl.ds((j0 + k) * LANES, LANES)]
                      for k in range(GROUP)]
                for k in range(GROUP):
                    plsc.addupdate(
                        gb.at[r, pl.ds((j0 + k) * LANES, LANES)], xs[k])
            return carry
        lax.fori_loop(0, SUBR, row_body, 0)

    pending_g = {0: gcopy(0), 1: gcopy(1)}
    out_cp = {}
    for c in range(NCH):
        bg = c % 3
        if c + 2 < NCH:
            if c >= 1:
                for cp in out_cp.pop(c - 1):
                    cp.wait()
            pending_g[c + 2] = gcopy(c + 2)
        pending_g.pop(c).wait()
        if c == 0:
            p_cp.wait()
        off = (c % HALVES) * CH
        row0 = (c // HALVES) * S + (c % HALVES) * CH + s_base
        cps = []
        for sub in range(NSUB):
            gb = gbuf[bg].at[pl.ds(sub * SUBR, SUBR)]
            add_sub(gb, pbuf.at[pl.ds(off + sub * SUBR, SUBR)])
            cps.append(pltpu.async_copy(
                gb, out_hbm.at[pl.ds(row0 + sub * SUBR, SUBR)], osem[bg]))
        out_cp[c] = cps
    for c in sorted(out_cp):
        for cp in out_cp.pop(c):
            cp.wait()


@functools.partial(
    pl.kernel,
    mesh=plsc.VectorSubcoreMesh(core_axis_name="c", subcore_axis_name="s"),
    out_type=jax.ShapeDtypeStruct((NROWS, EMBED), jnp.float32),
    scratch_types=[
        pltpu.VMEM((NCH, CH), jnp.int32),
        pltpu.VMEM((CH, EMBED), jnp.float32),
        pltpu.VMEM((CH, EMBED), jnp.float32),
        pltpu.VMEM((CH, EMBED), jnp.float32),
        pltpu.VMEM((SBLK, EMBED), jnp.float32),
        pltpu.SemaphoreType.DMA,
        pltpu.SemaphoreType.DMA,
        pltpu.SemaphoreType.DMA,
        pltpu.SemaphoreType.DMA,
        pltpu.SemaphoreType.DMA,
        pltpu.SemaphoreType.DMA,
        pltpu.SemaphoreType.DMA,
        pltpu.SemaphoreType.DMA,
    ],
)
def _emb(ids_hbm, wte_hbm, wpe_hbm, out_hbm, *scratch):
    _emb_body(ids_hbm, wte_hbm, wpe_hbm, out_hbm, *scratch)


def kernel(input_ids, wte, wpe):
    batch, seq = input_ids.shape
    ids4 = input_ids.astype(jnp.int32).reshape(batch, NW, HALVES, CH)
    out = _emb(ids4, wte, wpe)
    return out.reshape(batch, seq, EMBED)


# speedup vs baseline: 1.1971x; 1.0824x over previous
import functools

import jax
import jax.numpy as jnp
from jax import lax
from jax.experimental import pallas as pl
from jax.experimental.pallas import tpu as pltpu
from jax.experimental.pallas import tpu_sc as plsc

EMBED = 768
B, S = 4, 2048
NROWS = B * S

NC, NS = 2, 16
NW = NC * NS
SBLK = S // NW
CHS = 16
NSC = SBLK // CHS
ROWS = B * CHS
LANES = 16
VECS = EMBED // LANES
GROUP = 24


def _emb_body(ids_hbm, wte_hbm, wpe_hbm, out_hbm,
              idx_v, g0, g1, p0, p1,
              gs0, gs1, ps0, ps1, os0, os1, isem):
    wid = lax.axis_index("s") * NC + lax.axis_index("c")
    s_base = wid * SBLK

    idx_cp = [pltpu.async_copy(ids_hbm.at[b, wid],
                               idx_v.at[pl.ds(b * NSC, NSC)], isem)
              for b in range(B)]
    for cp in idx_cp:
        cp.wait()

    gbuf = (g0, g1)
    pbuf = (p0, p1)
    gsem = (gs0, gs1)
    psem = (ps0, ps1)
    osem = (os0, os1)

    def gissue(x):
        bg = x % 2
        return [pltpu.async_copy(wte_hbm.at[idx_v.at[b * NSC + x]],
                                 gbuf[bg].at[pl.ds(b * CHS, CHS)],
                                 gsem[bg])
                for b in range(B)]

    def pissue(x):
        bp = x % 2
        return pltpu.async_copy(
            wpe_hbm.at[pl.ds(s_base + x * CHS, CHS)], pbuf[bp], psem[bp])

    def add_super(bg, bp):
        def row_body(r, carry):
            for j0 in range(0, VECS, GROUP):
                xs = [pbuf[bp][r, pl.ds((j0 + k) * LANES, LANES)]
                      for k in range(GROUP)]
                for b in range(B):
                    for k in range(GROUP):
                        plsc.addupdate(
                            gbuf[bg].at[b * CHS + r,
                                        pl.ds((j0 + k) * LANES, LANES)],
                            xs[k])
            return carry
        lax.fori_loop(0, CHS, row_body, 0)

    def oissue(x):
        bg = x % 2
        return [pltpu.async_copy(gbuf[bg].at[pl.ds(b * CHS, CHS)],
                                 out_hbm.at[pl.ds(b * S + s_base + x * CHS,
                                                  CHS)],
                                 osem[bg])
                for b in range(B)]

    pending_g = {0: gissue(0)}
    pending_p = {0: pissue(0)}
    out_cp = {}
    for x in range(NSC):
        bg, bp = x % 2, x % 2
        if x + 1 < NSC:
            if x >= 1:
                for cp in out_cp.pop(x - 1):
                    cp.wait()
            pending_g[x + 1] = gissue(x + 1)
            pending_p[x + 1] = pissue(x + 1)
        for cp in pending_g.pop(x):
            cp.wait()
        pending_p.pop(x).wait()
        add_super(bg, bp)
        out_cp[x] = oissue(x)
    for x in sorted(out_cp):
        for cp in out_cp.pop(x):
            cp.wait()


@functools.partial(
    pl.kernel,
    mesh=plsc.VectorSubcoreMesh(core_axis_name="c", subcore_axis_name="s"),
    out_type=jax.ShapeDtypeStruct((NROWS, EMBED), jnp.float32),
    scratch_types=[
        pltpu.VMEM((B * NSC, CHS), jnp.int32),
        pltpu.VMEM((ROWS, EMBED), jnp.float32),
        pltpu.VMEM((ROWS, EMBED), jnp.float32),
        pltpu.VMEM((CHS, EMBED), jnp.float32),
        pltpu.VMEM((CHS, EMBED), jnp.float32),
        pltpu.SemaphoreType.DMA,
        pltpu.SemaphoreType.DMA,
        pltpu.SemaphoreType.DMA,
        pltpu.SemaphoreType.DMA,
        pltpu.SemaphoreType.DMA,
        pltpu.SemaphoreType.DMA,
        pltpu.SemaphoreType.DMA,
    ],
)
def _emb(ids_hbm, wte_hbm, wpe_hbm, out_hbm, *scratch):
    _emb_body(ids_hbm, wte_hbm, wpe_hbm, out_hbm, *scratch)


def kernel(input_ids, wte, wpe):
    batch, seq = input_ids.shape
    ids4 = input_ids.astype(jnp.int32).reshape(batch, NW, NSC, CHS)
    out = _emb(ids4, wte, wpe)
    return out.reshape(batch, seq, EMBED)
